# baseline (speedup 1.0000x reference)
import jax
import jax.numpy as jnp
from jax import lax
from jax.experimental import pallas as pl
from jax.experimental.pallas import tpu as pltpu
from jax.experimental.pallas import tpu_sc as plsc

VOCAB = 1000000
MAXLEN = 200
EMBDIM = 64
BATCH = 4096
SEQ = 200

NUM_CORES = 2
NUM_SUBCORES = 16
NUM_WORKERS = NUM_CORES * NUM_SUBCORES
SEQ_PER_WORKER = BATCH // NUM_WORKERS
ROWS_PER_CHUNK = SEQ
NCHUNK = SEQ_PER_WORKER
NBUF = 7
GD = 4
ROWS_PER_WORKER = SEQ_PER_WORKER * SEQ


def _tokposemb_body(x_hbm, tok_hbm, pos_hbm, out_hbm, pos_sh, scratch):
    buf_v = scratch["buf"]
    idx_bufs = scratch["idx"]
    sem_i, sem_p, sem_g, sem_w = (scratch[k] for k in ("si", "sp", "sg", "sw"))

    sid = lax.axis_index("s")
    wid = sid * NUM_CORES + lax.axis_index("c")
    row_base = wid * ROWS_PER_WORKER
    seq_base = wid * SEQ_PER_WORKER

    @pl.when(sid == 0)
    def _():
        pltpu.sync_copy(pos_hbm, buf_v.at[0, pl.ds(0, MAXLEN)])
        pltpu.sync_copy(buf_v.at[0, pl.ds(0, MAXLEN)], pos_sh)

    plsc.subcore_barrier()

    def chunk_rows(i):
        return row_base + i * ROWS_PER_CHUNK

    def issue_idx(i, b):
        pltpu.async_copy(x_hbm.at[seq_base + i], idx_bufs[b], sem_i.at[b])

    def wait_idx(i, b):
        pltpu.make_async_copy(x_hbm.at[seq_base + i], idx_bufs[b],
                              sem_i.at[b]).wait()

    def issue_posfill(b):
        pltpu.async_copy(pos_sh, buf_v.at[b], sem_p.at[b])

    def wait_posfill(b):
        pltpu.make_async_copy(pos_sh, buf_v.at[b], sem_p.at[b]).wait()

    def issue_gather(b):
        pltpu.async_copy(tok_hbm.at[idx_bufs[b]], buf_v.at[b], sem_g.at[b],
                         add=True)

    def wait_gather(b):
        pltpu.make_async_copy(tok_hbm.at[idx_bufs[b]], buf_v.at[b],
                              sem_g.at[b]).wait()

    def issue_wb(i, b):
        pltpu.async_copy(buf_v.at[b], out_hbm.at[seq_base + i], sem_w.at[b])

    def wait_wb(i, b):
        pltpu.make_async_copy(buf_v.at[b], out_hbm.at[seq_base + i],
                              sem_w.at[b]).wait()

    n_steps = NCHUNK + GD + 1
    n_outer = n_steps // NBUF

    def outer(g, carry):
        for b_off in range(NBUF):
            j = g * NBUF + b_off

            @pl.when(j < NCHUNK)
            def _():
                @pl.when(j >= NBUF)
                def _():
                    wait_wb(j - NBUF, b_off)
                issue_idx(j, b_off)
                issue_posfill(b_off)

            jg = j - 1
            bg = (b_off - 1) % NBUF

            @pl.when(jnp.logical_and(jg >= 0, jg < NCHUNK))
            def _():
                wait_idx(jg, bg)
                wait_posfill(bg)
                issue_gather(bg)

            jr = j - 1 - GD
            br = (b_off - 1 - GD) % NBUF

            @pl.when(jnp.logical_and(jr >= 0, jr < NCHUNK))
            def _():
                wait_gather(br)
                issue_wb(jr, br)
        return carry

    lax.fori_loop(0, n_outer, outer, 0)

    for j in range(NCHUNK - NBUF, NCHUNK):
        wait_wb(j, j % NBUF)


def _body(x_hbm, tok_hbm, pos_hbm, out_hbm, pos_sh, buf_v,
          i0, i1, i2, i3, i4, i5, i6, sem_i, sem_p, sem_g, sem_w):
    scratch = {
        "buf": buf_v,
        "idx": [i0, i1, i2, i3, i4, i5, i6],
        "si": sem_i, "sp": sem_p, "sg": sem_g, "sw": sem_w,
    }
    _tokposemb_body(x_hbm, tok_hbm, pos_hbm, out_hbm, pos_sh, scratch)


@jax.jit
def _tokposemb(x_flat, tok_table, pos_table):
    mesh = plsc.VectorSubcoreMesh(core_axis_name="c", subcore_axis_name="s")
    return pl.kernel(
        _body,
        out_type=jax.ShapeDtypeStruct((BATCH, SEQ, EMBDIM), jnp.float32),
        mesh=mesh,
        scratch_types=[
            pltpu.VMEM_SHARED((ROWS_PER_CHUNK, EMBDIM), jnp.float32),
            pltpu.VMEM((NBUF, ROWS_PER_CHUNK, EMBDIM), jnp.float32),
        ] + [pltpu.VMEM((ROWS_PER_CHUNK,), jnp.int32) for _ in range(NBUF)] + [
            pltpu.SemaphoreType.DMA((NBUF,)),
            pltpu.SemaphoreType.DMA((NBUF,)),
            pltpu.SemaphoreType.DMA((NBUF,)),
            pltpu.SemaphoreType.DMA((NBUF,)),
        ],
        compiler_params=pltpu.CompilerParams(use_tc_tiling_on_sc=False),
    )(x_flat, tok_table, pos_table)


def kernel(x, tok_table, pos_table):
    return _tokposemb(x.astype(jnp.int32), tok_table, pos_table)


# speedup vs baseline: 1.5153x; 1.5153x over previous
import functools

import jax
import jax.numpy as jnp
from jax.experimental.layout import Format, Layout, with_layout_constraint
from jax import lax
from jax.experimental import pallas as pl
from jax.experimental.pallas import tpu as pltpu
from jax.experimental.pallas import tpu_sc as plsc

VOCAB = 1000000
MAXLEN = 200
EMBDIM = 64
BATCH = 4096
SEQ = 200

NUM_CORES = 2
NUM_SUBCORES = 16
NUM_WORKERS = NUM_CORES * NUM_SUBCORES
SEQ_PER_WORKER = BATCH // NUM_WORKERS
ROWS_PER_CHUNK = SEQ
NCHUNK = SEQ_PER_WORKER
NBUF = 7
GD = 4
ROWS_PER_WORKER = SEQ_PER_WORKER * SEQ


def _tokposemb_body(x_hbm, tok_hbm, pos_hbm, out_hbm, pos_sh, scratch):
    buf_v = scratch["buf"]
    idx_bufs = scratch["idx"]
    sem_i, sem_p, sem_g, sem_w = (scratch[k] for k in ("si", "sp", "sg", "sw"))

    sid = lax.axis_index("s")
    wid = sid * NUM_CORES + lax.axis_index("c")
    row_base = wid * ROWS_PER_WORKER
    seq_base = wid * SEQ_PER_WORKER

    @pl.when(sid == 0)
    def _():
        pltpu.sync_copy(pos_hbm, buf_v.at[0, pl.ds(0, MAXLEN)])
        pltpu.sync_copy(buf_v.at[0, pl.ds(0, MAXLEN)], pos_sh)

    plsc.subcore_barrier()

    def chunk_rows(i):
        return row_base + i * ROWS_PER_CHUNK

    def issue_idx(i, b):
        pltpu.async_copy(x_hbm.at[seq_base + i], idx_bufs[b], sem_i.at[b])

    def wait_idx(i, b):
        pltpu.make_async_copy(x_hbm.at[seq_base + i], idx_bufs[b],
                              sem_i.at[b]).wait()

    def issue_posfill(b):
        pltpu.async_copy(pos_sh, buf_v.at[b], sem_p.at[b])

    def wait_posfill(b):
        pltpu.make_async_copy(pos_sh, buf_v.at[b], sem_p.at[b]).wait()

    def issue_gather(b):
        pltpu.async_copy(tok_hbm.at[idx_bufs[b]], buf_v.at[b], sem_g.at[b],
                         add=True)

    def wait_gather(b):
        pltpu.make_async_copy(tok_hbm.at[idx_bufs[b]], buf_v.at[b],
                              sem_g.at[b]).wait()

    def issue_wb(i, b):
        pltpu.async_copy(buf_v.at[b], out_hbm.at[seq_base + i], sem_w.at[b])

    def wait_wb(i, b):
        pltpu.make_async_copy(buf_v.at[b], out_hbm.at[seq_base + i],
                              sem_w.at[b]).wait()

    n_steps = NCHUNK + GD + 1
    n_outer = n_steps // NBUF

    def outer(g, carry):
        for b_off in range(NBUF):
            j = g * NBUF + b_off

            @pl.when(j < NCHUNK)
            def _():
                @pl.when(j >= NBUF)
                def _():
                    wait_wb(j - NBUF, b_off)
                issue_idx(j, b_off)
                issue_posfill(b_off)

            jg = j - 1
            bg = (b_off - 1) % NBUF

            @pl.when(jnp.logical_and(jg >= 0, jg < NCHUNK))
            def _():
                wait_idx(jg, bg)
                wait_posfill(bg)
                issue_gather(bg)

            jr = j - 1 - GD
            br = (b_off - 1 - GD) % NBUF

            @pl.when(jnp.logical_and(jr >= 0, jr < NCHUNK))
            def _():
                wait_gather(br)
                issue_wb(jr, br)
        return carry

    lax.fori_loop(0, n_outer, outer, 0)

    for j in range(NCHUNK - NBUF, NCHUNK):
        wait_wb(j, j % NBUF)


def _body(x_hbm, tok_hbm, pos_hbm, out_hbm, pos_sh, buf_v,
          i0, i1, i2, i3, i4, i5, i6, sem_i, sem_p, sem_g, sem_w):
    scratch = {
        "buf": buf_v,
        "idx": [i0, i1, i2, i3, i4, i5, i6],
        "si": sem_i, "sp": sem_p, "sg": sem_g, "sw": sem_w,
    }
    _tokposemb_body(x_hbm, tok_hbm, pos_hbm, out_hbm, pos_sh, scratch)


def _tokposemb(x_flat, tok_table, pos_table):
    mesh = plsc.VectorSubcoreMesh(core_axis_name="c", subcore_axis_name="s")
    return pl.kernel(
        _body,
        out_type=jax.ShapeDtypeStruct((BATCH, SEQ, EMBDIM), jnp.float32),
        mesh=mesh,
        scratch_types=[
            pltpu.VMEM_SHARED((ROWS_PER_CHUNK, EMBDIM), jnp.float32),
            pltpu.VMEM((NBUF, ROWS_PER_CHUNK, EMBDIM), jnp.float32),
        ] + [pltpu.VMEM((ROWS_PER_CHUNK,), jnp.int32) for _ in range(NBUF)] + [
            pltpu.SemaphoreType.DMA((NBUF,)),
            pltpu.SemaphoreType.DMA((NBUF,)),
            pltpu.SemaphoreType.DMA((NBUF,)),
            pltpu.SemaphoreType.DMA((NBUF,)),
        ],
        compiler_params=pltpu.CompilerParams(use_tc_tiling_on_sc=False),
    )(x_flat, tok_table, pos_table)


def kernel(x, tok_table, pos_table):
    tok_table = with_layout_constraint(
        tok_table, Layout(major_to_minor=(0, 1), tiling=((8, 64),)))
    out = _tokposemb(x.astype(jnp.int32), tok_table, pos_table)
    return with_layout_constraint(
        out, Layout(major_to_minor=(0, 1, 2), tiling=((8, 64),)))


# speedup vs baseline: 2.4348x; 1.6068x over previous
import functools

import jax
import jax.numpy as jnp
from jax.experimental.layout import Format, Layout, with_layout_constraint
from jax import lax
from jax.experimental import pallas as pl
from jax.experimental.pallas import tpu as pltpu
from jax.experimental.pallas import tpu_sc as plsc

VOCAB = 1000000
MAXLEN = 200
EMBDIM = 64
BATCH = 4096
SEQ = 200

NUM_CORES = 2
NUM_SUBCORES = 16
NUM_WORKERS = NUM_CORES * NUM_SUBCORES
SEQ_PER_WORKER = BATCH // NUM_WORKERS
ROWS_PER_CHUNK = SEQ
NCHUNK = SEQ_PER_WORKER
NBUF = 7
GD = 4
ROWS_PER_WORKER = SEQ_PER_WORKER * SEQ


def _tokposemb_body(x_hbm, tok_hbm, pos_hbm, out_hbm, pos_sh, scratch):
    buf_v = scratch["buf"]
    idx_bufs = scratch["idx"]
    sem_i, sem_p, sem_g, sem_w = (scratch[k] for k in ("si", "sp", "sg", "sw"))

    sid = lax.axis_index("s")
    wid = sid * NUM_CORES + lax.axis_index("c")
    row_base = wid * ROWS_PER_WORKER
    seq_base = wid * SEQ_PER_WORKER

    @pl.when(sid == 0)
    def _():
        pltpu.sync_copy(pos_hbm, buf_v.at[0, pl.ds(0, MAXLEN)])
        pltpu.sync_copy(buf_v.at[0, pl.ds(0, MAXLEN)], pos_sh)

    plsc.subcore_barrier()

    def chunk_rows(i):
        return row_base + i * ROWS_PER_CHUNK

    def issue_idx(i, b):
        pltpu.async_copy(x_hbm.at[seq_base + i], idx_bufs[b], sem_i.at[b])

    def wait_idx(i, b):
        pltpu.make_async_copy(x_hbm.at[seq_base + i], idx_bufs[b],
                              sem_i.at[b]).wait()

    def issue_posfill(b):
        pltpu.async_copy(pos_sh, buf_v.at[b], sem_p.at[b])

    def wait_posfill(b):
        pltpu.make_async_copy(pos_sh, buf_v.at[b], sem_p.at[b]).wait()

    def issue_gather(b):
        pltpu.async_copy(tok_hbm.at[idx_bufs[b]], buf_v.at[b], sem_g.at[b],
                         add=True)

    def wait_gather(b):
        pltpu.make_async_copy(tok_hbm.at[idx_bufs[b]], buf_v.at[b],
                              sem_g.at[b]).wait()

    def issue_wb(i, b):
        pltpu.async_copy(buf_v.at[b],
                         out_hbm.at[seq_base + i, :, pl.ds(0, EMBDIM)],
                         sem_w.at[b])

    def wait_wb(i, b):
        pltpu.make_async_copy(buf_v.at[b],
                              out_hbm.at[seq_base + i, :, pl.ds(0, EMBDIM)],
                              sem_w.at[b]).wait()

    n_steps = NCHUNK + GD + 1
    n_outer = n_steps // NBUF

    def outer(g, carry):
        for b_off in range(NBUF):
            j = g * NBUF + b_off

            @pl.when(j < NCHUNK)
            def _():
                @pl.when(j >= NBUF)
                def _():
                    wait_wb(j - NBUF, b_off)
                issue_idx(j, b_off)
                issue_posfill(b_off)

            jg = j - 1
            bg = (b_off - 1) % NBUF

            @pl.when(jnp.logical_and(jg >= 0, jg < NCHUNK))
            def _():
                wait_idx(jg, bg)
                wait_posfill(bg)
                issue_gather(bg)

            jr = j - 1 - GD
            br = (b_off - 1 - GD) % NBUF

            @pl.when(jnp.logical_and(jr >= 0, jr < NCHUNK))
            def _():
                wait_gather(br)
                issue_wb(jr, br)
        return carry

    lax.fori_loop(0, n_outer, outer, 0)

    for j in range(NCHUNK - NBUF, NCHUNK):
        wait_wb(j, j % NBUF)


def _body(x_hbm, tok_hbm, pos_hbm, out_hbm, pos_sh, buf_v,
          i0, i1, i2, i3, i4, i5, i6, sem_i, sem_p, sem_g, sem_w):
    scratch = {
        "buf": buf_v,
        "idx": [i0, i1, i2, i3, i4, i5, i6],
        "si": sem_i, "sp": sem_p, "sg": sem_g, "sw": sem_w,
    }
    _tokposemb_body(x_hbm, tok_hbm, pos_hbm, out_hbm, pos_sh, scratch)


def _tokposemb(x_flat, tok_table, pos_table):
    mesh = plsc.VectorSubcoreMesh(core_axis_name="c", subcore_axis_name="s")
    return pl.kernel(
        _body,
        out_type=jax.ShapeDtypeStruct((BATCH, SEQ, 2 * EMBDIM), jnp.float32),
        mesh=mesh,
        scratch_types=[
            pltpu.VMEM_SHARED((ROWS_PER_CHUNK, EMBDIM), jnp.float32),
            pltpu.VMEM((NBUF, ROWS_PER_CHUNK, EMBDIM), jnp.float32),
        ] + [pltpu.VMEM((ROWS_PER_CHUNK,), jnp.int32) for _ in range(NBUF)] + [
            pltpu.SemaphoreType.DMA((NBUF,)),
            pltpu.SemaphoreType.DMA((NBUF,)),
            pltpu.SemaphoreType.DMA((NBUF,)),
            pltpu.SemaphoreType.DMA((NBUF,)),
        ],
        compiler_params=pltpu.CompilerParams(use_tc_tiling_on_sc=False),
    )(x_flat, tok_table, pos_table)


def kernel(x, tok_table, pos_table):
    tok_table = with_layout_constraint(
        tok_table, Layout(major_to_minor=(0, 1), tiling=((8, 64),)))
    out_pad = _tokposemb(x.astype(jnp.int32), tok_table, pos_table)
    out = jax.lax.slice(out_pad, (0, 0, 0), (BATCH, SEQ, EMBDIM))
    return with_layout_constraint(
        out, Layout(major_to_minor=(0, 1, 2), tiling=((8, 128),)))
